# baseline (speedup 1.0000x reference)
import functools

import jax
import jax.numpy as jnp
from jax import lax
from jax.experimental import pallas as pl
from jax.experimental.pallas import tpu as pltpu
from jax.experimental.pallas import tpu_sc as plsc

B = 16384
F = 16
L = 16
NC = 2
NS = 16
NW = NC * NS
BPW = B // NW
CH = 128
NCHUNK = BPW // CH


def _sc_body(user_r, item_r, uf, itf, uint_r, iint_r, g_r, out_r,
             uidx, iidx, urows, irows, uintv, iintv, outv, gv, sem):
    c = lax.axis_index("c")
    s = lax.axis_index("s")
    wid = s * NC + c
    base = wid * BPW

    pltpu.sync_copy(user_r.at[wid], uidx)
    pltpu.sync_copy(item_r.at[wid], iidx)
    pltpu.sync_copy(g_r, gv)

    copies = []
    for ci in range(NCHUNK):
        sl = pl.ds(ci * CH, CH)
        copies.append(pltpu.async_copy(uf.at[uidx.at[ci]], urows.at[sl], sem))
        copies.append(pltpu.async_copy(itf.at[iidx.at[ci]], irows.at[sl], sem))
        copies.append(pltpu.async_copy(uint_r.at[uidx.at[ci]], uintv.at[sl], sem))
        copies.append(pltpu.async_copy(iint_r.at[iidx.at[ci]], iintv.at[sl], sem))
    for cp in copies:
        cp.wait()

    iota = lax.iota(jnp.int32, L)
    gvec = gv[...]

    def tile_body(t, carry):
        r0 = t * L
        acc = uintv[pl.ds(r0, L)] + iintv[pl.ds(r0, L)] + gvec
        for j in range(L):
            p = urows[r0 + j, :] * irows[r0 + j, :]
            s = jnp.sum(p)
            acc = jnp.where(iota == j, acc + s, acc)
        outv[pl.ds(r0, L)] = acc
        return carry

    lax.fori_loop(0, BPW // L, tile_body, 0)

    pltpu.sync_copy(outv, out_r.at[pl.ds(base, BPW)])


@functools.partial(
    pl.kernel,
    mesh=plsc.VectorSubcoreMesh(core_axis_name="c", subcore_axis_name="s"),
    out_type=jax.ShapeDtypeStruct((B,), jnp.float32),
    compiler_params=pltpu.CompilerParams(
        needs_layout_passes=False, use_tc_tiling_on_sc=False),
    scratch_types=[
        pltpu.VMEM((NCHUNK, CH), jnp.int32),
        pltpu.VMEM((NCHUNK, CH), jnp.int32),
        pltpu.VMEM((BPW, F), jnp.float32),
        pltpu.VMEM((BPW, F), jnp.float32),
        pltpu.VMEM((BPW,), jnp.float32),
        pltpu.VMEM((BPW,), jnp.float32),
        pltpu.VMEM((BPW,), jnp.float32),
        pltpu.VMEM((L,), jnp.float32),
        pltpu.SemaphoreType.DMA,
    ],
)
def _sc_kernel(*refs):
    _sc_body(*refs)


def kernel(user, item, user_factors, item_factors, user_intercepts,
           item_intercepts, global_intercept):
    user_r = user.reshape(NW, NCHUNK, CH)
    item_r = item.reshape(NW, NCHUNK, CH)
    uint_r = user_intercepts.reshape(-1)
    iint_r = item_intercepts.reshape(-1)
    g_r = jnp.broadcast_to(global_intercept.reshape(()), (L,))
    return _sc_kernel(user_r, item_r, user_factors, item_factors,
                      uint_r, iint_r, g_r)
